# baseline (speedup 1.0000x reference)
import dataclasses
import functools

import numpy as np

import jax
import jax.numpy as jnp
from jax import lax
from jax.experimental import pallas as pl
from jax.experimental.pallas import tpu as pltpu
from jax.experimental.pallas import tpu_sc as plsc

NC, NS, L = 2, 16, 16
NW = NC * NS

TW = 384
GCH = 40
SCH = 80

_PREC = lax.Precision.DEFAULT


def _silu(v):
    return v * jax.nn.sigmoid(v)


def _prep_tables(h, x, Wea, Web):
    N, D = h.shape
    NB = 2000

    def body(h_ref, x_ref, wa_ref, wb_ref, a_ref, b_ref):
        hb = h_ref[...]
        xp = jnp.concatenate(
            [x_ref[...], jnp.zeros((NB, 125), jnp.float32)], axis=1)
        x3 = xp.astype(jnp.bfloat16)
        zz = jnp.zeros((NB, 128), jnp.bfloat16)
        a = jnp.dot(hb, wa_ref[...], preferred_element_type=jnp.float32,
                    precision=_PREC)
        b = jnp.dot(hb, wb_ref[...], preferred_element_type=jnp.float32,
                    precision=_PREC)
        a_ref[...] = jnp.concatenate([a.astype(jnp.bfloat16), x3, zz], axis=1)
        b_ref[...] = jnp.concatenate([b.astype(jnp.bfloat16), x3, zz], axis=1)

    return pl.pallas_call(
        body,
        grid=(N // NB,),
        in_specs=[
            pl.BlockSpec((NB, D), lambda i: (i, 0)),
            pl.BlockSpec((NB, 3), lambda i: (i, 0)),
            pl.BlockSpec((D, D), lambda i: (0, 0)),
            pl.BlockSpec((D, D), lambda i: (0, 0)),
        ],
        out_specs=[
            pl.BlockSpec((NB, 512), lambda i: (i, 0)),
            pl.BlockSpec((NB, 512), lambda i: (i, 0)),
        ],
        out_shape=[
            jax.ShapeDtypeStruct((N, 512), jnp.bfloat16),
            jax.ShapeDtypeStruct((N, 512), jnp.bfloat16),
        ],
    )(h, x, Wea, Web)


def _gather_tables(At, Bt, ridx, cidx, E):
    mesh = plsc.VectorSubcoreMesh(core_axis_name="c", subcore_axis_name="s")
    EPW = E // NW
    NCHK = EPW // GCH
    D = TW - 128
    assert NCHK % 2 == 1 and NCHK >= 3

    cp = pltpu.CompilerParams()
    if "needs_layout_passes" in pltpu.CompilerParams.__dataclass_fields__:
        cp = dataclasses.replace(cp, needs_layout_passes=False)

    @functools.partial(
        pl.kernel,
        out_type=jax.ShapeDtypeStruct((E, TW), jnp.float32),
        mesh=mesh,
        compiler_params=cp,
        scratch_types=[
            pltpu.VMEM((NCHK, GCH), jnp.int32),
            pltpu.VMEM((NCHK, GCH), jnp.int32),
            pltpu.VMEM((GCH, 256), jnp.int32),
            pltpu.VMEM((GCH, 256), jnp.int32),
            pltpu.VMEM((GCH, 256), jnp.int32),
            pltpu.VMEM((GCH, 256), jnp.int32),
            pltpu.VMEM((GCH, TW), jnp.float32),
            pltpu.VMEM((GCH, TW), jnp.float32),
            pltpu.SemaphoreType.DMA,
            pltpu.SemaphoreType.DMA,
            pltpu.SemaphoreType.DMA,
            pltpu.SemaphoreType.DMA,
        ],
    )
    def gk(a_hbm, b_hbm, ri_hbm, ci_hbm, g_hbm,
           ri_v, ci_v, a0, b0, a1, b1, o0, o1, sg0, sg1, so0, so1):
        wid = lax.axis_index("s") * NC + lax.axis_index("c")
        base = wid * EPW
        pltpu.sync_copy(ri_hbm.at[wid], ri_v)
        pltpu.sync_copy(ci_hbm.at[wid], ci_v)

        def g_pair(k, ab, bb, sg):
            return (pltpu.make_async_copy(a_hbm.at[ri_v.at[k]], ab, sg),
                    pltpu.make_async_copy(b_hbm.at[ci_v.at[k]], bb, sg))

        def o_copy(k, ob, so):
            dst = pl.ds(base + k * GCH, GCH)
            return pltpu.make_async_copy(ob, g_hbm.at[dst], so)

        def issue(pair):
            pair[0].start()
            pair[1].start()

        def wait(pair):
            pair[0].wait()
            pair[1].wait()

        MASK = jnp.int32(-65536)

        def deint_store(v, ob, e, c):
            w = plsc.bitcast(v, jnp.int32)
            ob[e, pl.ds(c, L)] = plsc.bitcast(w << 16, jnp.float32)
            ob[e, pl.ds(c + L, L)] = plsc.bitcast(w & MASK, jnp.float32)

        def bfp(buf, e, l):
            return plsc.bitcast(buf[e, pl.ds(l, L)], jnp.bfloat16)

        def fuse(ab, bb, ob):
            @pl.loop(0, GCH)
            def _(e):
                for gi in range(8):
                    s = bfp(ab, e, L * gi) + bfp(bb, e, L * gi)
                    deint_store(s, ob, e, 32 * gi)
                for gi in range(4):
                    d = bfp(ab, e, 128 + L * gi) - bfp(bb, e, 128 + L * gi)
                    deint_store(d, ob, e, D + 32 * gi)

        issue(g_pair(0, a0, b0, sg0))
        wait(g_pair(0, a0, b0, sg0))
        issue(g_pair(1, a1, b1, sg1))
        fuse(a0, b0, o0)
        o_copy(0, o0, so0).start()
        wait(g_pair(1, a1, b1, sg1))
        issue(g_pair(2, a0, b0, sg0))
        fuse(a1, b1, o1)
        o_copy(1, o1, so1).start()

        @pl.loop(2, NCHK - 2, step=2)
        def _(k):
            wait(g_pair(k, a0, b0, sg0))
            issue(g_pair(k + 1, a1, b1, sg1))
            o_copy(k - 2, o0, so0).wait()
            fuse(a0, b0, o0)
            o_copy(k, o0, so0).start()
            wait(g_pair(k + 1, a1, b1, sg1))
            issue(g_pair(k + 2, a0, b0, sg0))
            o_copy(k - 1, o1, so1).wait()
            fuse(a1, b1, o1)
            o_copy(k + 1, o1, so1).start()

        k = NCHK - 1
        wait(g_pair(k, a0, b0, sg0))
        o_copy(k - 2, o0, so0).wait()
        fuse(a0, b0, o0)
        o_copy(k, o0, so0).start()
        o_copy(k - 1, o1, so1).wait()
        o_copy(k, o0, so0).wait()

    return gk(At, Bt, ridx.reshape(NW, NCHK, GCH), cidx.reshape(NW, NCHK, GCH))


def _edge_mlp(G, edge_attr, wc, Wd, be1, We2, be2):
    E = G.shape[0]
    D = We2.shape[0]
    ED = Wd.shape[0]
    EB = 2000

    def body(g_ref, ea_ref, wc_ref, wd_ref, b1_ref, w2_ref, b2_ref,
             m_ref):
        g = g_ref[...]
        dx = g[:, D:TW]
        radial = jnp.sum(dx * dx, axis=1, keepdims=True)
        lin = (g[:, :D] + radial * wc_ref[...]
               + jnp.dot(ea_ref[...], wd_ref[...],
                         preferred_element_type=jnp.float32, precision=_PREC)
               + b1_ref[...])
        m1 = _silu(lin)
        m2 = jnp.dot(m1, w2_ref[...], preferred_element_type=jnp.float32,
                     precision=_PREC) + b2_ref[...]
        m_ref[...] = _silu(m2)

    return pl.pallas_call(
        body,
        grid=(E // EB,),
        in_specs=[
            pl.BlockSpec((EB, TW), lambda i: (i, 0)),
            pl.BlockSpec((EB, ED), lambda i: (i, 0)),
            pl.BlockSpec((1, D), lambda i: (0, 0)),
            pl.BlockSpec((ED, D), lambda i: (0, 0)),
            pl.BlockSpec((1, D), lambda i: (0, 0)),
            pl.BlockSpec((D, D), lambda i: (0, 0)),
            pl.BlockSpec((1, D), lambda i: (0, 0)),
        ],
        out_specs=pl.BlockSpec((EB, D), lambda i: (i, 0)),
        out_shape=jax.ShapeDtypeStruct((E, D), jnp.float32),
    )(G, edge_attr, wc, Wd, be1.reshape(1, D), We2, be2.reshape(1, D))


def _segment_sum(m, ridx, N):
    E, D = m.shape
    DH = D // NC
    EPS = E // NS
    NCH = EPS // SCH
    assert NCH % 2 == 1 and NCH >= 3
    ZB = 1000
    NZ = N // ZB
    mesh = plsc.VectorSubcoreMesh(core_axis_name="c", subcore_axis_name="s")

    ridx3 = ridx.reshape(NS, NCH, SCH)
    zeros = jnp.zeros((N, DH), jnp.float32)

    @functools.partial(
        pl.kernel,
        out_type=jax.ShapeDtypeStruct((N, D), jnp.float32),
        mesh=mesh,
        scratch_types=[
            pltpu.VMEM((NCH, SCH), jnp.int32),
            pltpu.VMEM((SCH, DH), jnp.float32),
            pltpu.VMEM((SCH, DH), jnp.float32),
            pltpu.VMEM_SHARED((N, DH), jnp.float32),
            pltpu.SemaphoreType.DMA,
            pltpu.SemaphoreType.DMA,
        ],
    )
    def sk(m_hbm, ri_hbm, z_hbm, agg_hbm, idx_v, buf0, buf1, table, s0, s1):
        c = lax.axis_index("c")
        s = lax.axis_index("s")
        col0 = c * DH
        @pl.when(s < NZ)
        def _():
            pltpu.sync_copy(z_hbm.at[pl.ds(s * ZB, ZB)],
                            table.at[pl.ds(s * ZB, ZB)])
        pltpu.sync_copy(ri_hbm.at[s], idx_v)
        plsc.subcore_barrier()

        def load(k, buf, sem):
            return pltpu.make_async_copy(
                m_hbm.at[pl.ds(s * EPS + k * SCH, SCH), pl.ds(col0, DH)],
                buf, sem)

        def scat(k, buf):
            pltpu.sync_copy(buf, table.at[idx_v.at[k]], add=True)

        load(0, buf0, s0).start()

        @pl.loop(0, NCH - 1, step=2)
        def _(k):
            load(k + 1, buf1, s1).start()
            load(k, buf0, s0).wait()
            scat(k, buf0)
            load(k + 2, buf0, s0).start()
            load(k + 1, buf1, s1).wait()
            scat(k + 1, buf1)

        load(NCH - 1, buf0, s0).wait()
        scat(NCH - 1, buf0)

        plsc.subcore_barrier()
        @pl.when(s < NZ)
        def _():
            pltpu.sync_copy(table.at[pl.ds(s * ZB, ZB)],
                            agg_hbm.at[pl.ds(s * ZB, ZB), pl.ds(col0, DH)])

    return sk(m, ridx3, zeros)


def _node_mlp(h, agg, Wn1h, Wn1a, bn1, Wn2, bn2):
    N, D = h.shape
    NB = 2000

    P = len(agg)

    def body(h_ref, *refs):
        g_refs = refs[:P]
        w1h_ref, w1a_ref, b1_ref, w2_ref, b2_ref, o_ref = refs[P:]
        hb = h_ref[...]
        g = g_refs[0][...]
        for gr in g_refs[1:]:
            g = g + gr[...]
        lin = (jnp.dot(hb, w1h_ref[...], preferred_element_type=jnp.float32,
                       precision=_PREC)
               + jnp.dot(g, w1a_ref[...],
                         preferred_element_type=jnp.float32, precision=_PREC)
               + b1_ref[...])
        t = _silu(lin)
        o_ref[...] = (jnp.dot(t, w2_ref[...],
                              preferred_element_type=jnp.float32,
                              precision=_PREC)
                      + b2_ref[...] + hb)

    return pl.pallas_call(
        body,
        grid=(N // NB,),
        in_specs=[pl.BlockSpec((NB, D), lambda i: (i, 0))]
        + [pl.BlockSpec((NB, D), lambda i: (i, 0)) for _ in range(P)]
        + [
            pl.BlockSpec((D, D), lambda i: (0, 0)),
            pl.BlockSpec((D, D), lambda i: (0, 0)),
            pl.BlockSpec((1, D), lambda i: (0, 0)),
            pl.BlockSpec((D, D), lambda i: (0, 0)),
            pl.BlockSpec((1, D), lambda i: (0, 0)),
        ],
        out_specs=pl.BlockSpec((NB, D), lambda i: (i, 0)),
        out_shape=jax.ShapeDtypeStruct((N, D), jnp.float32),
    )(h, *agg, Wn1h, Wn1a, bn1.reshape(1, D), Wn2, bn2.reshape(1, D))


def kernel(h, x, edges, edge_attr, We1, be1, We2, be2, Wn1, bn1, Wn2, bn2):
    N, D = h.shape
    E = edges.shape[1]
    row = edges[0].astype(jnp.int32)
    col = edges[1].astype(jnp.int32)

    Wea = We1[0:D]
    Web = We1[D:2 * D]
    wc = We1[2 * D:2 * D + 1]
    Wd = We1[2 * D + 1:]

    og = np.arange(D).reshape(-1, 32)
    og = np.concatenate([og[:, 0::2], og[:, 1::2]], axis=1).reshape(-1)

    At, Bt = _prep_tables(h, x, Wea, Web)
    At = lax.bitcast_convert_type(At.reshape(N, 256, 2), jnp.int32)
    Bt = lax.bitcast_convert_type(Bt.reshape(N, 256, 2), jnp.int32)
    G = _gather_tables(At, Bt, row, col, E)
    m = _edge_mlp(G, edge_attr, wc[:, og], Wd[:, og], be1[og], We2[og, :],
                  be2)
    aggs = [_segment_sum(m, row, N)]
    h_new = _node_mlp(h, aggs, Wn1[0:D], Wn1[D:2 * D], bn1, Wn2, bn2)
    return (h_new, x)


# speedup vs baseline: 1.6789x; 1.6789x over previous
import functools

import jax
import jax.numpy as jnp
from jax import lax
from jax.experimental import pallas as pl
from jax.experimental.pallas import tpu as pltpu
from jax.experimental.pallas import tpu_sc as plsc

NC, NS, L = 2, 16, 16
NW = NC * NS

TW = 384
GCH = 40
SCH = 40

_PREC = lax.Precision.DEFAULT


def _silu(v):
    return v * jax.nn.sigmoid(v)


def _prep_tables(h, x, Wea, Web):
    N, D = h.shape
    NB = 2000

    def body(h_ref, x_ref, wa_ref, wb_ref, a_ref, b_ref):
        hb = h_ref[...]
        xp = jnp.concatenate(
            [x_ref[...], jnp.zeros((NB, TW - D - 3), jnp.float32)], axis=1)
        a = jnp.dot(hb, wa_ref[...], preferred_element_type=jnp.float32,
                    precision=_PREC)
        b = jnp.dot(hb, wb_ref[...], preferred_element_type=jnp.float32,
                    precision=_PREC)
        a_ref[...] = jnp.concatenate([a, xp], axis=1)
        b_ref[...] = jnp.concatenate([b, xp], axis=1)

    return pl.pallas_call(
        body,
        grid=(N // NB,),
        in_specs=[
            pl.BlockSpec((NB, D), lambda i: (i, 0)),
            pl.BlockSpec((NB, 3), lambda i: (i, 0)),
            pl.BlockSpec((D, D), lambda i: (0, 0)),
            pl.BlockSpec((D, D), lambda i: (0, 0)),
        ],
        out_specs=[
            pl.BlockSpec((NB, TW), lambda i: (i, 0)),
            pl.BlockSpec((NB, TW), lambda i: (i, 0)),
        ],
        out_shape=[
            jax.ShapeDtypeStruct((N, TW), jnp.float32),
            jax.ShapeDtypeStruct((N, TW), jnp.float32),
        ],
    )(h, x, Wea, Web)


def _gather_tables(At, Bt, ridx, cidx, E):
    mesh = plsc.VectorSubcoreMesh(core_axis_name="c", subcore_axis_name="s")
    EPW = E // NW
    NCHK = EPW // GCH
    D = TW - 128
    assert NCHK % 2 == 1 and NCHK >= 3

    @functools.partial(
        pl.kernel,
        out_type=jax.ShapeDtypeStruct((E, TW), jnp.float32),
        mesh=mesh,
        scratch_types=[
            pltpu.VMEM((NCHK, GCH), jnp.int32),
            pltpu.VMEM((NCHK, GCH), jnp.int32),
            pltpu.VMEM((GCH, TW), jnp.float32),
            pltpu.VMEM((GCH, TW), jnp.float32),
            pltpu.VMEM((GCH, TW), jnp.float32),
            pltpu.VMEM((GCH, TW), jnp.float32),
            pltpu.VMEM((GCH, TW), jnp.float32),
            pltpu.VMEM((GCH, TW), jnp.float32),
            pltpu.SemaphoreType.DMA,
            pltpu.SemaphoreType.DMA,
            pltpu.SemaphoreType.DMA,
            pltpu.SemaphoreType.DMA,
        ],
    )
    def gk(a_hbm, b_hbm, ri_hbm, ci_hbm, g_hbm,
           ri_v, ci_v, a0, b0, a1, b1, o0, o1, sg0, sg1, so0, so1):
        wid = lax.axis_index("s") * NC + lax.axis_index("c")
        base = wid * EPW
        pltpu.sync_copy(ri_hbm.at[wid], ri_v)
        pltpu.sync_copy(ci_hbm.at[wid], ci_v)

        def g_pair(k, ab, bb, sg):
            return (pltpu.make_async_copy(a_hbm.at[ri_v.at[k]], ab, sg),
                    pltpu.make_async_copy(b_hbm.at[ci_v.at[k]], bb, sg))

        def o_copy(k, ob, so):
            dst = pl.ds(base + k * GCH, GCH)
            return pltpu.make_async_copy(ob, g_hbm.at[dst], so)

        def issue(pair):
            pair[0].start()
            pair[1].start()

        def wait(pair):
            pair[0].wait()
            pair[1].wait()

        def fuse(ab, bb, ob):
            @pl.loop(0, GCH)
            def _(e):
                for j in range(0, D, L):
                    ob[e, pl.ds(j, L)] = (ab[e, pl.ds(j, L)]
                                          + bb[e, pl.ds(j, L)])
                for j in range(D, TW, L):
                    ob[e, pl.ds(j, L)] = (ab[e, pl.ds(j, L)]
                                          - bb[e, pl.ds(j, L)])

        issue(g_pair(0, a0, b0, sg0))
        wait(g_pair(0, a0, b0, sg0))
        issue(g_pair(1, a1, b1, sg1))
        fuse(a0, b0, o0)
        o_copy(0, o0, so0).start()
        wait(g_pair(1, a1, b1, sg1))
        issue(g_pair(2, a0, b0, sg0))
        fuse(a1, b1, o1)
        o_copy(1, o1, so1).start()

        @pl.loop(2, NCHK - 2, step=2)
        def _(k):
            wait(g_pair(k, a0, b0, sg0))
            issue(g_pair(k + 1, a1, b1, sg1))
            o_copy(k - 2, o0, so0).wait()
            fuse(a0, b0, o0)
            o_copy(k, o0, so0).start()
            wait(g_pair(k + 1, a1, b1, sg1))
            issue(g_pair(k + 2, a0, b0, sg0))
            o_copy(k - 1, o1, so1).wait()
            fuse(a1, b1, o1)
            o_copy(k + 1, o1, so1).start()

        k = NCHK - 1
        wait(g_pair(k, a0, b0, sg0))
        o_copy(k - 2, o0, so0).wait()
        fuse(a0, b0, o0)
        o_copy(k, o0, so0).start()
        o_copy(k - 1, o1, so1).wait()
        o_copy(k, o0, so0).wait()

    return gk(At, Bt, ridx.reshape(NW, NCHK, GCH), cidx.reshape(NW, NCHK, GCH))


def _edge_mlp(G, edge_attr, wc, Wd, be1, We2, be2, e0, ne):
    D = We2.shape[0]
    ED = Wd.shape[0]
    EB = 2000
    OB = e0 // EB

    def body(g_ref, ea_ref, wc_ref, wd_ref, b1_ref, w2_ref, b2_ref,
             m_ref):
        g = g_ref[...]
        dx = g[:, D:TW]
        radial = jnp.sum(dx * dx, axis=1, keepdims=True)
        lin = (g[:, :D] + radial * wc_ref[...]
               + jnp.dot(ea_ref[...], wd_ref[...],
                         preferred_element_type=jnp.float32, precision=_PREC)
               + b1_ref[...])
        m1 = _silu(lin)
        m2 = jnp.dot(m1, w2_ref[...], preferred_element_type=jnp.float32,
                     precision=_PREC) + b2_ref[...]
        m_ref[...] = _silu(m2)

    return pl.pallas_call(
        body,
        grid=(ne // EB,),
        in_specs=[
            pl.BlockSpec((EB, TW), lambda i: (i + OB, 0)),
            pl.BlockSpec((EB, ED), lambda i: (i + OB, 0)),
            pl.BlockSpec((1, D), lambda i: (0, 0)),
            pl.BlockSpec((ED, D), lambda i: (0, 0)),
            pl.BlockSpec((1, D), lambda i: (0, 0)),
            pl.BlockSpec((D, D), lambda i: (0, 0)),
            pl.BlockSpec((1, D), lambda i: (0, 0)),
        ],
        out_specs=pl.BlockSpec((EB, D), lambda i: (i, 0)),
        out_shape=jax.ShapeDtypeStruct((ne, D), jnp.float32),
    )(G, edge_attr, wc, Wd, be1.reshape(1, D), We2, be2.reshape(1, D))


def _segment_sum(m, ridx, N):
    E, D = m.shape
    DH = D // NC
    EPS = E // NS
    NCH = EPS // SCH
    assert NCH % 2 == 1 and NCH >= 3
    ZB = 1000
    NZ = N // ZB
    mesh = plsc.VectorSubcoreMesh(core_axis_name="c", subcore_axis_name="s")

    ridx3 = ridx.reshape(NS, NCH, SCH)
    zeros = jnp.zeros((N, DH), jnp.float32)

    @functools.partial(
        pl.kernel,
        out_type=jax.ShapeDtypeStruct((N, D), jnp.float32),
        mesh=mesh,
        scratch_types=[
            pltpu.VMEM((NCH, SCH), jnp.int32),
            pltpu.VMEM((SCH, DH), jnp.float32),
            pltpu.VMEM((SCH, DH), jnp.float32),
            pltpu.VMEM_SHARED((N, DH), jnp.float32),
            pltpu.SemaphoreType.DMA,
            pltpu.SemaphoreType.DMA,
        ],
    )
    def sk(m_hbm, ri_hbm, z_hbm, agg_hbm, idx_v, buf0, buf1, table, s0, s1):
        c = lax.axis_index("c")
        s = lax.axis_index("s")
        col0 = c * DH
        @pl.when(s < NZ)
        def _():
            pltpu.sync_copy(z_hbm.at[pl.ds(s * ZB, ZB)],
                            table.at[pl.ds(s * ZB, ZB)])
        pltpu.sync_copy(ri_hbm.at[s], idx_v)
        plsc.subcore_barrier()

        def load(k, buf, sem):
            return pltpu.make_async_copy(
                m_hbm.at[pl.ds(s * EPS + k * SCH, SCH), pl.ds(col0, DH)],
                buf, sem)

        def scat(k, buf):
            pltpu.sync_copy(buf, table.at[idx_v.at[k]], add=True)

        load(0, buf0, s0).start()

        @pl.loop(0, NCH - 1, step=2)
        def _(k):
            load(k + 1, buf1, s1).start()
            load(k, buf0, s0).wait()
            scat(k, buf0)
            load(k + 2, buf0, s0).start()
            load(k + 1, buf1, s1).wait()
            scat(k + 1, buf1)

        load(NCH - 1, buf0, s0).wait()
        scat(NCH - 1, buf0)

        plsc.subcore_barrier()
        @pl.when(s < NZ)
        def _():
            pltpu.sync_copy(table.at[pl.ds(s * ZB, ZB)],
                            agg_hbm.at[pl.ds(s * ZB, ZB), pl.ds(col0, DH)])

    return sk(m, ridx3, zeros)


def _node_mlp(h, agg, Wn1h, Wn1a, bn1, Wn2, bn2):
    N, D = h.shape
    NB = 2000

    P = len(agg)

    def body(h_ref, *refs):
        g_refs = refs[:P]
        w1h_ref, w1a_ref, b1_ref, w2_ref, b2_ref, o_ref = refs[P:]
        hb = h_ref[...]
        g = g_refs[0][...]
        for gr in g_refs[1:]:
            g = g + gr[...]
        lin = (jnp.dot(hb, w1h_ref[...], preferred_element_type=jnp.float32,
                       precision=_PREC)
               + jnp.dot(g, w1a_ref[...],
                         preferred_element_type=jnp.float32, precision=_PREC)
               + b1_ref[...])
        t = _silu(lin)
        o_ref[...] = (jnp.dot(t, w2_ref[...],
                              preferred_element_type=jnp.float32,
                              precision=_PREC)
                      + b2_ref[...] + hb)

    return pl.pallas_call(
        body,
        grid=(N // NB,),
        in_specs=[pl.BlockSpec((NB, D), lambda i: (i, 0))]
        + [pl.BlockSpec((NB, D), lambda i: (i, 0)) for _ in range(P)]
        + [
            pl.BlockSpec((D, D), lambda i: (0, 0)),
            pl.BlockSpec((D, D), lambda i: (0, 0)),
            pl.BlockSpec((1, D), lambda i: (0, 0)),
            pl.BlockSpec((D, D), lambda i: (0, 0)),
            pl.BlockSpec((1, D), lambda i: (0, 0)),
        ],
        out_specs=pl.BlockSpec((NB, D), lambda i: (i, 0)),
        out_shape=jax.ShapeDtypeStruct((N, D), jnp.float32),
    )(h, *agg, Wn1h, Wn1a, bn1.reshape(1, D), Wn2, bn2.reshape(1, D))


def kernel(h, x, edges, edge_attr, We1, be1, We2, be2, Wn1, bn1, Wn2, bn2):
    N, D = h.shape
    E = edges.shape[1]
    row = edges[0].astype(jnp.int32)
    col = edges[1].astype(jnp.int32)

    Wea = We1[0:D]
    Web = We1[D:2 * D]
    wc = We1[2 * D:2 * D + 1]
    Wd = We1[2 * D + 1:]

    At, Bt = _prep_tables(h, x, Wea, Web)
    G = _gather_tables(At, Bt, row, col, E)
    half = E // 2
    aggs = []
    for p in range(2):
        m_p = _edge_mlp(G, edge_attr, wc, Wd, be1, We2, be2, p * half, half)
        aggs.append(_segment_sum(m_p, row[p * half:(p + 1) * half], N))
    h_new = _node_mlp(h, aggs, Wn1[0:D], Wn1[D:2 * D], bn1, Wn2, bn2)
    return (h_new, x)


# speedup vs baseline: 1.7166x; 1.0225x over previous
import functools

import jax
import jax.numpy as jnp
from jax import lax
from jax.experimental import pallas as pl
from jax.experimental.pallas import tpu as pltpu
from jax.experimental.pallas import tpu_sc as plsc

NC, NS, L = 2, 16, 16
NW = NC * NS

TW = 384
GCH = 40
SCH = 80

_PREC = lax.Precision.DEFAULT


def _silu(v):
    return v * jax.nn.sigmoid(v)


def _prep_tables(h, x, Wea, Web):
    N, D = h.shape
    NB = 2000

    def body(h_ref, x_ref, wa_ref, wb_ref, a_ref, b_ref):
        hb = h_ref[...]
        xp = jnp.concatenate(
            [x_ref[...], jnp.zeros((NB, TW - D - 3), jnp.float32)], axis=1)
        a = jnp.dot(hb, wa_ref[...], preferred_element_type=jnp.float32,
                    precision=_PREC)
        b = jnp.dot(hb, wb_ref[...], preferred_element_type=jnp.float32,
                    precision=_PREC)
        a_ref[...] = jnp.concatenate([a, xp], axis=1)
        b_ref[...] = jnp.concatenate([b, xp], axis=1)

    return pl.pallas_call(
        body,
        grid=(N // NB,),
        in_specs=[
            pl.BlockSpec((NB, D), lambda i: (i, 0)),
            pl.BlockSpec((NB, 3), lambda i: (i, 0)),
            pl.BlockSpec((D, D), lambda i: (0, 0)),
            pl.BlockSpec((D, D), lambda i: (0, 0)),
        ],
        out_specs=[
            pl.BlockSpec((NB, TW), lambda i: (i, 0)),
            pl.BlockSpec((NB, TW), lambda i: (i, 0)),
        ],
        out_shape=[
            jax.ShapeDtypeStruct((N, TW), jnp.float32),
            jax.ShapeDtypeStruct((N, TW), jnp.float32),
        ],
    )(h, x, Wea, Web)


def _gather_tables(At, Bt, ridx, cidx, E):
    mesh = plsc.VectorSubcoreMesh(core_axis_name="c", subcore_axis_name="s")
    EPW = E // NW
    NCHK = EPW // GCH
    D = TW - 128
    assert NCHK % 2 == 1 and NCHK >= 3

    @functools.partial(
        pl.kernel,
        out_type=jax.ShapeDtypeStruct((E, TW), jnp.float32),
        mesh=mesh,
        scratch_types=[
            pltpu.VMEM((NCHK, GCH), jnp.int32),
            pltpu.VMEM((NCHK, GCH), jnp.int32),
            pltpu.VMEM((GCH, TW), jnp.float32),
            pltpu.VMEM((GCH, TW), jnp.float32),
            pltpu.VMEM((GCH, TW), jnp.float32),
            pltpu.VMEM((GCH, TW), jnp.float32),
            pltpu.VMEM((GCH, TW), jnp.float32),
            pltpu.VMEM((GCH, TW), jnp.float32),
            pltpu.SemaphoreType.DMA,
            pltpu.SemaphoreType.DMA,
            pltpu.SemaphoreType.DMA,
            pltpu.SemaphoreType.DMA,
        ],
    )
    def gk(a_hbm, b_hbm, ri_hbm, ci_hbm, g_hbm,
           ri_v, ci_v, a0, b0, a1, b1, o0, o1, sg0, sg1, so0, so1):
        wid = lax.axis_index("s") * NC + lax.axis_index("c")
        base = wid * EPW
        pltpu.sync_copy(ri_hbm.at[wid], ri_v)
        pltpu.sync_copy(ci_hbm.at[wid], ci_v)

        def g_pair(k, ab, bb, sg):
            return (pltpu.make_async_copy(a_hbm.at[ri_v.at[k]], ab, sg),
                    pltpu.make_async_copy(b_hbm.at[ci_v.at[k]], bb, sg))

        def o_copy(k, ob, so):
            dst = pl.ds(base + k * GCH, GCH)
            return pltpu.make_async_copy(ob, g_hbm.at[dst], so)

        def issue(pair):
            pair[0].start()
            pair[1].start()

        def wait(pair):
            pair[0].wait()
            pair[1].wait()

        def fuse(ab, bb, ob):
            @pl.loop(0, GCH)
            def _(e):
                for j in range(0, D, L):
                    ob[e, pl.ds(j, L)] = (ab[e, pl.ds(j, L)]
                                          + bb[e, pl.ds(j, L)])
                for j in range(D, TW, L):
                    ob[e, pl.ds(j, L)] = (ab[e, pl.ds(j, L)]
                                          - bb[e, pl.ds(j, L)])

        issue(g_pair(0, a0, b0, sg0))
        wait(g_pair(0, a0, b0, sg0))
        issue(g_pair(1, a1, b1, sg1))
        fuse(a0, b0, o0)
        o_copy(0, o0, so0).start()
        wait(g_pair(1, a1, b1, sg1))
        issue(g_pair(2, a0, b0, sg0))
        fuse(a1, b1, o1)
        o_copy(1, o1, so1).start()

        @pl.loop(2, NCHK - 2, step=2)
        def _(k):
            wait(g_pair(k, a0, b0, sg0))
            issue(g_pair(k + 1, a1, b1, sg1))
            o_copy(k - 2, o0, so0).wait()
            fuse(a0, b0, o0)
            o_copy(k, o0, so0).start()
            wait(g_pair(k + 1, a1, b1, sg1))
            issue(g_pair(k + 2, a0, b0, sg0))
            o_copy(k - 1, o1, so1).wait()
            fuse(a1, b1, o1)
            o_copy(k + 1, o1, so1).start()

        k = NCHK - 1
        wait(g_pair(k, a0, b0, sg0))
        o_copy(k - 2, o0, so0).wait()
        fuse(a0, b0, o0)
        o_copy(k, o0, so0).start()
        o_copy(k - 1, o1, so1).wait()
        o_copy(k, o0, so0).wait()

    return gk(At, Bt, ridx.reshape(NW, NCHK, GCH), cidx.reshape(NW, NCHK, GCH))


def _edge_mlp(G, edge_attr, wc, Wd, be1, We2, be2):
    E = G.shape[0]
    D = We2.shape[0]
    ED = Wd.shape[0]
    EB = 2000

    def body(g_ref, ea_ref, wc_ref, wd_ref, b1_ref, w2_ref, b2_ref,
             m_ref):
        g = g_ref[...]
        dx = g[:, D:TW]
        radial = jnp.sum(dx * dx, axis=1, keepdims=True)
        lin = (g[:, :D] + radial * wc_ref[...]
               + jnp.dot(ea_ref[...], wd_ref[...],
                         preferred_element_type=jnp.float32, precision=_PREC)
               + b1_ref[...])
        m1 = _silu(lin)
        m2 = jnp.dot(m1, w2_ref[...], preferred_element_type=jnp.float32,
                     precision=_PREC) + b2_ref[...]
        m_ref[...] = _silu(m2)

    return pl.pallas_call(
        body,
        grid=(E // EB,),
        in_specs=[
            pl.BlockSpec((EB, TW), lambda i: (i, 0)),
            pl.BlockSpec((EB, ED), lambda i: (i, 0)),
            pl.BlockSpec((1, D), lambda i: (0, 0)),
            pl.BlockSpec((ED, D), lambda i: (0, 0)),
            pl.BlockSpec((1, D), lambda i: (0, 0)),
            pl.BlockSpec((D, D), lambda i: (0, 0)),
            pl.BlockSpec((1, D), lambda i: (0, 0)),
        ],
        out_specs=pl.BlockSpec((EB, D), lambda i: (i, 0)),
        out_shape=jax.ShapeDtypeStruct((E, D), jnp.float32),
    )(G, edge_attr, wc, Wd, be1.reshape(1, D), We2, be2.reshape(1, D))


def _segment_sum(m, ridx, N):
    E, D = m.shape
    DH = D // NC
    EPS = E // NS
    NCH = EPS // SCH
    assert NCH % 2 == 1 and NCH >= 3
    ZB = 1000
    NZ = N // ZB
    mesh = plsc.VectorSubcoreMesh(core_axis_name="c", subcore_axis_name="s")

    ridx3 = ridx.reshape(NS, NCH, SCH)
    zeros = jnp.zeros((N, DH), jnp.float32)

    @functools.partial(
        pl.kernel,
        out_type=jax.ShapeDtypeStruct((N, D), jnp.float32),
        mesh=mesh,
        scratch_types=[
            pltpu.VMEM((NCH, SCH), jnp.int32),
            pltpu.VMEM((SCH, DH), jnp.float32),
            pltpu.VMEM((SCH, DH), jnp.float32),
            pltpu.VMEM_SHARED((N, DH), jnp.float32),
            pltpu.SemaphoreType.DMA,
            pltpu.SemaphoreType.DMA,
        ],
    )
    def sk(m_hbm, ri_hbm, z_hbm, agg_hbm, idx_v, buf0, buf1, table, s0, s1):
        c = lax.axis_index("c")
        s = lax.axis_index("s")
        col0 = c * DH
        @pl.when(s < NZ)
        def _():
            pltpu.sync_copy(z_hbm.at[pl.ds(s * ZB, ZB)],
                            table.at[pl.ds(s * ZB, ZB)])
        pltpu.sync_copy(ri_hbm.at[s], idx_v)
        plsc.subcore_barrier()

        def load(k, buf, sem):
            return pltpu.make_async_copy(
                m_hbm.at[pl.ds(s * EPS + k * SCH, SCH), pl.ds(col0, DH)],
                buf, sem)

        def scat(k, buf):
            pltpu.sync_copy(buf, table.at[idx_v.at[k]], add=True)

        load(0, buf0, s0).start()

        @pl.loop(0, NCH - 1, step=2)
        def _(k):
            load(k + 1, buf1, s1).start()
            load(k, buf0, s0).wait()
            scat(k, buf0)
            load(k + 2, buf0, s0).start()
            load(k + 1, buf1, s1).wait()
            scat(k + 1, buf1)

        load(NCH - 1, buf0, s0).wait()
        scat(NCH - 1, buf0)

        plsc.subcore_barrier()
        @pl.when(s < NZ)
        def _():
            pltpu.sync_copy(table.at[pl.ds(s * ZB, ZB)],
                            agg_hbm.at[pl.ds(s * ZB, ZB), pl.ds(col0, DH)])

    return sk(m, ridx3, zeros)


def _node_mlp(h, agg, Wn1h, Wn1a, bn1, Wn2, bn2):
    N, D = h.shape
    NB = 2000

    P = len(agg)

    def body(h_ref, *refs):
        g_refs = refs[:P]
        w1h_ref, w1a_ref, b1_ref, w2_ref, b2_ref, o_ref = refs[P:]
        hb = h_ref[...]
        g = g_refs[0][...]
        for gr in g_refs[1:]:
            g = g + gr[...]
        lin = (jnp.dot(hb, w1h_ref[...], preferred_element_type=jnp.float32,
                       precision=_PREC)
               + jnp.dot(g, w1a_ref[...],
                         preferred_element_type=jnp.float32, precision=_PREC)
               + b1_ref[...])
        t = _silu(lin)
        o_ref[...] = (jnp.dot(t, w2_ref[...],
                              preferred_element_type=jnp.float32,
                              precision=_PREC)
                      + b2_ref[...] + hb)

    return pl.pallas_call(
        body,
        grid=(N // NB,),
        in_specs=[pl.BlockSpec((NB, D), lambda i: (i, 0))]
        + [pl.BlockSpec((NB, D), lambda i: (i, 0)) for _ in range(P)]
        + [
            pl.BlockSpec((D, D), lambda i: (0, 0)),
            pl.BlockSpec((D, D), lambda i: (0, 0)),
            pl.BlockSpec((1, D), lambda i: (0, 0)),
            pl.BlockSpec((D, D), lambda i: (0, 0)),
            pl.BlockSpec((1, D), lambda i: (0, 0)),
        ],
        out_specs=pl.BlockSpec((NB, D), lambda i: (i, 0)),
        out_shape=jax.ShapeDtypeStruct((N, D), jnp.float32),
    )(h, *agg, Wn1h, Wn1a, bn1.reshape(1, D), Wn2, bn2.reshape(1, D))


def kernel(h, x, edges, edge_attr, We1, be1, We2, be2, Wn1, bn1, Wn2, bn2):
    N, D = h.shape
    E = edges.shape[1]
    row = edges[0].astype(jnp.int32)
    col = edges[1].astype(jnp.int32)

    Wea = We1[0:D]
    Web = We1[D:2 * D]
    wc = We1[2 * D:2 * D + 1]
    Wd = We1[2 * D + 1:]

    At, Bt = _prep_tables(h, x, Wea, Web)
    G = _gather_tables(At, Bt, row, col, E)
    m = _edge_mlp(G, edge_attr, wc, Wd, be1, We2, be2)
    aggs = [_segment_sum(m, row, N)]
    h_new = _node_mlp(h, aggs, Wn1[0:D], Wn1[D:2 * D], bn1, Wn2, bn2)
    return (h_new, x)
